# baseline (speedup 1.0000x reference)
import functools

import numpy as np
import jax
import jax.numpy as jnp
from jax import lax
from jax.experimental import pallas as pl
from jax.experimental.pallas import tpu as pltpu
from jax.experimental.pallas import tpu_sc as plsc

H, W = 32, 64
N = H * W
NPAD = 2176
RB = 1088
NRB = NPAD // RB
BT = 8
FOV_UP_DEG, FOV_DOWN_DEG = 3.0, -25.0
MASK_THRESHOLD = 0.5
SENT = 1000.0


def _dirs_np():
    fov_up = FOV_UP_DEG * np.pi / 180.0
    fov_down = FOV_DOWN_DEG * np.pi / 180.0
    fov = abs(fov_up) + abs(fov_down)
    proj_y = (np.arange(H, dtype=np.float32) + 0.5) / H
    proj_x = (np.arange(W, dtype=np.float32) + 0.5) / W
    pitch = (1.0 - proj_y) * fov - abs(fov_down)
    yaw = (2.0 * proj_x - 1.0) * np.pi
    pitch = pitch[:, None]
    yaw = yaw[None, :]
    dx = np.cos(pitch) * np.cos(yaw)
    dy = np.cos(pitch) * np.sin(yaw)
    dz = np.sin(pitch) * np.ones_like(yaw)
    dirs = np.stack([np.broadcast_to(dx, (H, W)),
                     np.broadcast_to(dy, (H, W)),
                     np.broadcast_to(dz, (H, W))], axis=-1).astype(np.float32)
    return dirs.reshape(N, 3)


_DIRS = _dirs_np()


def _build_body(out_ref, mask_ref, tr_ref, tx_ref, ty_ref, tz_ref,
                dx_ref, dy_ref, dz_ref,
                ax_ref, ay_ref, az_ref, bx_ref, by_ref, bz_ref):
    r = jnp.where(mask_ref[...] > MASK_THRESHOLD, out_ref[...], -1.0)
    valid = r > 0.0
    ax = jnp.where(valid, r * dx_ref[...], SENT)
    ay = jnp.where(valid, r * dy_ref[...], SENT)
    az = jnp.where(valid, r * dz_ref[...], SENT)
    tvalid = tr_ref[...] >= 0.0
    bx = jnp.where(tvalid, tx_ref[...], SENT)
    by = jnp.where(tvalid, ty_ref[...], SENT)
    bz = jnp.where(tvalid, tz_ref[...], SENT)
    for dst, src in ((ax_ref, ax), (ay_ref, ay), (az_ref, az),
                     (bx_ref, bx), (by_ref, by), (bz_ref, bz)):
        dst[:, :N] = src
        dst[:, N:] = jnp.full((BT, NPAD - N), SENT, jnp.float32)


def _pair_body(axc, ayc, azc, bxr, byr, bzr, out_ref, colmin, acc):
    rb = pl.program_id(1)
    a_x = axc[0]
    a_y = ayc[0]
    a_z = azc[0]
    b_x = bxr[0]
    b_y = byr[0]
    b_z = bzr[0]
    dx = a_x - b_x
    dy = a_y - b_y
    dz = a_z - b_z
    d = dx * dx + dy * dy + dz * dz
    rmin = jnp.min(d, axis=1)
    s1 = jnp.sum(rmin)
    c1 = jnp.sum((rmin > 0.0).astype(jnp.float32))
    cm = jnp.min(d, axis=0, keepdims=True)

    @pl.when(rb == 0)
    def _():
        colmin[...] = cm
        acc[0] = s1
        acc[1] = c1

    @pl.when(rb > 0)
    def _():
        colmin[...] = jnp.minimum(colmin[...], cm)
        acc[0] = acc[0] + s1
        acc[1] = acc[1] + c1

    @pl.when(rb == NRB - 1)
    def _():
        cmf = colmin[...]
        s2 = jnp.sum(cmf)
        c2 = jnp.sum((cmf > 0.0).astype(jnp.float32))
        out_ref[...] = jnp.full((1, 1, 1), acc[0] / acc[1] + s2 / c2,
                                jnp.float32)


def _build_points(out2, mask2, tr, tx, ty, tz):
    dx = _DIRS[:, 0].reshape(1, N)
    dy = _DIRS[:, 1].reshape(1, N)
    dz = _DIRS[:, 2].reshape(1, N)
    plane = jax.ShapeDtypeStruct((BT, NPAD), jnp.float32)
    return pl.pallas_call(
        _build_body,
        out_shape=(plane,) * 6,
    )(out2, mask2, tr, tx, ty, tz,
      jnp.asarray(dx), jnp.asarray(dy), jnp.asarray(dz))


def _pairwise(ax, ay, az, bx, by, bz):
    nf = BT - _NSC
    a_spec = pl.BlockSpec((1, RB, 1), lambda f, rb: (f + _NSC, rb, 0))
    b_spec = pl.BlockSpec((1, 1, NPAD), lambda f, rb: (f + _NSC, 0, 0))
    return pl.pallas_call(
        _pair_body,
        grid=(nf, NRB),
        in_specs=[a_spec, a_spec, a_spec, b_spec, b_spec, b_spec],
        out_specs=pl.BlockSpec((1, 1, 1), lambda f, rb: (f, 0, 0)),
        out_shape=jax.ShapeDtypeStruct((nf, 1, 1), jnp.float32),
        scratch_shapes=[
            pltpu.VMEM((1, NPAD), jnp.float32),
            pltpu.SMEM((2,), jnp.float32),
        ],
    )(ax.reshape(BT, NPAD, 1), ay.reshape(BT, NPAD, 1),
      az.reshape(BT, NPAD, 1),
      bx.reshape(BT, 1, NPAD), by.reshape(BT, 1, NPAD),
      bz.reshape(BT, 1, NPAD))


_NSC = 8
_CPF = 32 // _NSC
_CH = {8: 528, 4: 272, 2: 144, 1: 80}[_NSC]
_NPAD_SC = _CPF * _CH
_UB = {8: 3, 4: 1, 2: 3, 1: 5}[_NSC]
_NBLK = _CH // (16 * _UB)
_NBV = _NPAD_SC // 16


_GDN = lax.GatherDimensionNumbers(
    offset_dims=(), collapsed_slice_dims=(0,), start_index_map=(0,))


def _lane_splat(v, l):
    idx = jnp.full((16, 1), l, jnp.int32)
    return lax.gather(v, idx, _GDN, (1,),
                      mode=lax.GatherScatterMode.PROMISE_IN_BOUNDS)


def _sc_pass(cx, cy, cz, fx, fy, fz):
    big = jnp.full((16,), 1e30, jnp.float32)

    def outer(blk, carry):
        sacc, cacc = carry
        base = blk * (16 * _UB)
        avs = [(cx[pl.ds(base + 16 * u, 16)],
                cy[pl.ds(base + 16 * u, 16)],
                cz[pl.ds(base + 16 * u, 16)]) for u in range(_UB)]

        def inner(j16, ms):
            jb = j16 * 16
            bxv = fx[pl.ds(jb, 16)]
            byv = fy[pl.ds(jb, 16)]
            bzv = fz[pl.ds(jb, 16)]
            out = list(ms)
            for l in range(16):
                bxs = _lane_splat(bxv, l)
                bys = _lane_splat(byv, l)
                bzs = _lane_splat(bzv, l)
                for u in range(_UB):
                    ddx = avs[u][0] - bxs
                    ddy = avs[u][1] - bys
                    ddz = avs[u][2] - bzs
                    dd = ddx * ddx + ddy * ddy + ddz * ddz
                    out[u] = jnp.minimum(out[u], dd)
            return tuple(out)

        ms = lax.fori_loop(0, _NBV, inner, (big,) * _UB)
        for u in range(_UB):
            sacc = sacc + ms[u]
            cacc = cacc + jnp.sign(ms[u])
        return sacc, cacc

    zero = jnp.zeros((16,), jnp.float32)
    return lax.fori_loop(0, _NBLK, outer, (zero, zero))


def _sc_body(ax_h, ay_h, az_h, bx_h, by_h, bz_h, out_h,
             cx, cy, cz, fx, fy, fz, ost):
    c = lax.axis_index("c")
    s = lax.axis_index("s")
    w = s * 2 + c
    f = w // _CPF
    q = w % _CPF
    frow = f * _NPAD_SC
    coff = frow + q * _CH
    pltpu.sync_copy(ax_h.at[pl.ds(coff, _CH)], cx)
    pltpu.sync_copy(ay_h.at[pl.ds(coff, _CH)], cy)
    pltpu.sync_copy(az_h.at[pl.ds(coff, _CH)], cz)
    pltpu.sync_copy(bx_h.at[pl.ds(frow, _NPAD_SC)], fx)
    pltpu.sync_copy(by_h.at[pl.ds(frow, _NPAD_SC)], fy)
    pltpu.sync_copy(bz_h.at[pl.ds(frow, _NPAD_SC)], fz)
    s1, c1 = _sc_pass(cx, cy, cz, fx, fy, fz)
    pltpu.sync_copy(bx_h.at[pl.ds(coff, _CH)], cx)
    pltpu.sync_copy(by_h.at[pl.ds(coff, _CH)], cy)
    pltpu.sync_copy(bz_h.at[pl.ds(coff, _CH)], cz)
    pltpu.sync_copy(ax_h.at[pl.ds(frow, _NPAD_SC)], fx)
    pltpu.sync_copy(ay_h.at[pl.ds(frow, _NPAD_SC)], fy)
    pltpu.sync_copy(az_h.at[pl.ds(frow, _NPAD_SC)], fz)
    s2, c2 = _sc_pass(cx, cy, cz, fx, fy, fz)
    ost[pl.ds(0, 16)] = s1
    ost[pl.ds(16, 16)] = c1
    ost[pl.ds(32, 16)] = s2
    ost[pl.ds(48, 16)] = c2
    pltpu.sync_copy(ost, out_h.at[pl.ds(w * 64, 64)])


def _sc_chamfer(ax, ay, az, bx, by, bz):
    mesh = plsc.VectorSubcoreMesh(core_axis_name="c", subcore_axis_name="s")
    run = functools.partial(
        pl.kernel,
        out_type=jax.ShapeDtypeStruct((32 * 64,), jnp.float32),
        mesh=mesh,
        scratch_types=[
            pltpu.VMEM((_CH,), jnp.float32),
            pltpu.VMEM((_CH,), jnp.float32),
            pltpu.VMEM((_CH,), jnp.float32),
            pltpu.VMEM((_NPAD_SC,), jnp.float32),
            pltpu.VMEM((_NPAD_SC,), jnp.float32),
            pltpu.VMEM((_NPAD_SC,), jnp.float32),
            pltpu.VMEM((64,), jnp.float32),
        ],
    )(_sc_body)
    partials = run(ax.reshape(-1), ay.reshape(-1), az.reshape(-1),
                   bx.reshape(-1), by.reshape(-1), bz.reshape(-1))
    p = partials.reshape(_NSC, _CPF, 4, 16).sum(axis=(1, 3))
    return p[:, 0] / p[:, 1] + p[:, 2] / p[:, 3]


def _sc_planes(plane):
    sub = plane[:_NSC]
    if _NPAD_SC <= NPAD:
        return sub[:, :_NPAD_SC]
    return jnp.pad(sub, ((0, 0), (0, _NPAD_SC - NPAD)),
                   constant_values=SENT)


def kernel(output, mask, target):
    B, T = output.shape[0], output.shape[1]
    out2 = output.reshape(BT, N)
    mask2 = mask.reshape(BT, N)
    tr = target[:, :, 0].reshape(BT, N)
    tx = target[:, :, 1].reshape(BT, N)
    ty = target[:, :, 2].reshape(BT, N)
    tz = target[:, :, 3].reshape(BT, N)
    ax, ay, az, bx, by, bz = _build_points(out2, mask2, tr, tx, ty, tz)
    parts = []
    if _NSC > 0:
        parts.append(_sc_chamfer(*(_sc_planes(p)
                                   for p in (ax, ay, az, bx, by, bz))))
    if _NSC < BT:
        parts.append(_pairwise(ax, ay, az, bx, by, bz).reshape(BT - _NSC))
    dc = jnp.concatenate(parts) if len(parts) > 1 else parts[0]
    ct = dc.reshape(T, B)
    return (jnp.mean(ct, axis=1), ct)


# speedup vs baseline: 14.1061x; 14.1061x over previous
import functools

import numpy as np
import jax
import jax.numpy as jnp
from jax import lax
from jax.experimental import pallas as pl
from jax.experimental.pallas import tpu as pltpu
from jax.experimental.pallas import tpu_sc as plsc

H, W = 32, 64
N = H * W
NPAD = 2176
RB = 1088
NRB = NPAD // RB
BT = 8
FOV_UP_DEG, FOV_DOWN_DEG = 3.0, -25.0
MASK_THRESHOLD = 0.5
SENT = 1000.0


def _dirs_np():
    fov_up = FOV_UP_DEG * np.pi / 180.0
    fov_down = FOV_DOWN_DEG * np.pi / 180.0
    fov = abs(fov_up) + abs(fov_down)
    proj_y = (np.arange(H, dtype=np.float32) + 0.5) / H
    proj_x = (np.arange(W, dtype=np.float32) + 0.5) / W
    pitch = (1.0 - proj_y) * fov - abs(fov_down)
    yaw = (2.0 * proj_x - 1.0) * np.pi
    pitch = pitch[:, None]
    yaw = yaw[None, :]
    dx = np.cos(pitch) * np.cos(yaw)
    dy = np.cos(pitch) * np.sin(yaw)
    dz = np.sin(pitch) * np.ones_like(yaw)
    dirs = np.stack([np.broadcast_to(dx, (H, W)),
                     np.broadcast_to(dy, (H, W)),
                     np.broadcast_to(dz, (H, W))], axis=-1).astype(np.float32)
    return dirs.reshape(N, 3)


_DIRS = _dirs_np()


def _build_body(out_ref, mask_ref, tr_ref, tx_ref, ty_ref, tz_ref,
                dx_ref, dy_ref, dz_ref,
                ax_ref, ay_ref, az_ref, bx_ref, by_ref, bz_ref):
    r = jnp.where(mask_ref[...] > MASK_THRESHOLD, out_ref[...], -1.0)
    valid = r > 0.0
    ax = jnp.where(valid, r * dx_ref[...], SENT)
    ay = jnp.where(valid, r * dy_ref[...], SENT)
    az = jnp.where(valid, r * dz_ref[...], SENT)
    tvalid = tr_ref[...] >= 0.0
    bx = jnp.where(tvalid, tx_ref[...], SENT)
    by = jnp.where(tvalid, ty_ref[...], SENT)
    bz = jnp.where(tvalid, tz_ref[...], SENT)
    for dst, src in ((ax_ref, ax), (ay_ref, ay), (az_ref, az),
                     (bx_ref, bx), (by_ref, by), (bz_ref, bz)):
        dst[:, :N] = src
        dst[:, N:] = jnp.full((BT, NPAD - N), SENT, jnp.float32)


def _pair_body(axc, ayc, azc, bxr, byr, bzr, out_ref, colmin, acc):
    rb = pl.program_id(1)
    a_x = axc[0]
    a_y = ayc[0]
    a_z = azc[0]
    b_x = bxr[0]
    b_y = byr[0]
    b_z = bzr[0]
    dx = a_x - b_x
    dy = a_y - b_y
    dz = a_z - b_z
    d = dx * dx + dy * dy + dz * dz
    rmin = jnp.min(d, axis=1)
    s1 = jnp.sum(rmin)
    c1 = jnp.sum((rmin > 0.0).astype(jnp.float32))
    cm = jnp.min(d, axis=0, keepdims=True)

    @pl.when(rb == 0)
    def _():
        colmin[...] = cm
        acc[0] = s1
        acc[1] = c1

    @pl.when(rb > 0)
    def _():
        colmin[...] = jnp.minimum(colmin[...], cm)
        acc[0] = acc[0] + s1
        acc[1] = acc[1] + c1

    @pl.when(rb == NRB - 1)
    def _():
        cmf = colmin[...]
        s2 = jnp.sum(cmf)
        c2 = jnp.sum((cmf > 0.0).astype(jnp.float32))
        out_ref[...] = jnp.full((1, 1, 1), acc[0] / acc[1] + s2 / c2,
                                jnp.float32)


def _build_points(out2, mask2, tr, tx, ty, tz):
    dx = _DIRS[:, 0].reshape(1, N)
    dy = _DIRS[:, 1].reshape(1, N)
    dz = _DIRS[:, 2].reshape(1, N)
    plane = jax.ShapeDtypeStruct((BT, NPAD), jnp.float32)
    return pl.pallas_call(
        _build_body,
        out_shape=(plane,) * 6,
    )(out2, mask2, tr, tx, ty, tz,
      jnp.asarray(dx), jnp.asarray(dy), jnp.asarray(dz))


def _pairwise(ax, ay, az, bx, by, bz):
    nf = BT - _NSC
    a_spec = pl.BlockSpec((1, RB, 1), lambda f, rb: (f + _NSC, rb, 0))
    b_spec = pl.BlockSpec((1, 1, NPAD), lambda f, rb: (f + _NSC, 0, 0))
    return pl.pallas_call(
        _pair_body,
        grid=(nf, NRB),
        in_specs=[a_spec, a_spec, a_spec, b_spec, b_spec, b_spec],
        out_specs=pl.BlockSpec((1, 1, 1), lambda f, rb: (f, 0, 0)),
        out_shape=jax.ShapeDtypeStruct((nf, 1, 1), jnp.float32),
        scratch_shapes=[
            pltpu.VMEM((1, NPAD), jnp.float32),
            pltpu.SMEM((2,), jnp.float32),
        ],
    )(ax.reshape(BT, NPAD, 1), ay.reshape(BT, NPAD, 1),
      az.reshape(BT, NPAD, 1),
      bx.reshape(BT, 1, NPAD), by.reshape(BT, 1, NPAD),
      bz.reshape(BT, 1, NPAD))


_NSC = 0
_CPF = 32 // max(_NSC, 1)
_CH = {8: 528, 4: 272, 2: 144, 1: 80}.get(_NSC, 16)
_NPAD_SC = _CPF * _CH
_UB = {8: 3, 4: 1, 2: 3, 1: 5}.get(_NSC, 1)
_NBLK = _CH // (16 * _UB)
_NBV = _NPAD_SC // 16


_GDN = lax.GatherDimensionNumbers(
    offset_dims=(), collapsed_slice_dims=(0,), start_index_map=(0,))


def _lane_splat(v, l):
    idx = jnp.full((16, 1), l, jnp.int32)
    return lax.gather(v, idx, _GDN, (1,),
                      mode=lax.GatherScatterMode.PROMISE_IN_BOUNDS)


def _sc_pass(cx, cy, cz, fx, fy, fz):
    big = jnp.full((16,), 1e30, jnp.float32)

    def outer(blk, carry):
        sacc, cacc = carry
        base = blk * (16 * _UB)
        avs = [(cx[pl.ds(base + 16 * u, 16)],
                cy[pl.ds(base + 16 * u, 16)],
                cz[pl.ds(base + 16 * u, 16)]) for u in range(_UB)]

        def inner(j16, ms):
            jb = j16 * 16
            bxv = fx[pl.ds(jb, 16)]
            byv = fy[pl.ds(jb, 16)]
            bzv = fz[pl.ds(jb, 16)]
            out = list(ms)
            for l in range(16):
                bxs = _lane_splat(bxv, l)
                bys = _lane_splat(byv, l)
                bzs = _lane_splat(bzv, l)
                for u in range(_UB):
                    ddx = avs[u][0] - bxs
                    ddy = avs[u][1] - bys
                    ddz = avs[u][2] - bzs
                    dd = ddx * ddx + ddy * ddy + ddz * ddz
                    out[u] = jnp.minimum(out[u], dd)
            return tuple(out)

        ms = lax.fori_loop(0, _NBV, inner, (big,) * _UB)
        for u in range(_UB):
            sacc = sacc + ms[u]
            cacc = cacc + jnp.sign(ms[u])
        return sacc, cacc

    zero = jnp.zeros((16,), jnp.float32)
    return lax.fori_loop(0, _NBLK, outer, (zero, zero))


def _sc_body(ax_h, ay_h, az_h, bx_h, by_h, bz_h, out_h,
             cx, cy, cz, fx, fy, fz, ost):
    c = lax.axis_index("c")
    s = lax.axis_index("s")
    w = s * 2 + c
    f = w // _CPF
    q = w % _CPF
    frow = f * _NPAD_SC
    coff = frow + q * _CH
    pltpu.sync_copy(ax_h.at[pl.ds(coff, _CH)], cx)
    pltpu.sync_copy(ay_h.at[pl.ds(coff, _CH)], cy)
    pltpu.sync_copy(az_h.at[pl.ds(coff, _CH)], cz)
    pltpu.sync_copy(bx_h.at[pl.ds(frow, _NPAD_SC)], fx)
    pltpu.sync_copy(by_h.at[pl.ds(frow, _NPAD_SC)], fy)
    pltpu.sync_copy(bz_h.at[pl.ds(frow, _NPAD_SC)], fz)
    s1, c1 = _sc_pass(cx, cy, cz, fx, fy, fz)
    pltpu.sync_copy(bx_h.at[pl.ds(coff, _CH)], cx)
    pltpu.sync_copy(by_h.at[pl.ds(coff, _CH)], cy)
    pltpu.sync_copy(bz_h.at[pl.ds(coff, _CH)], cz)
    pltpu.sync_copy(ax_h.at[pl.ds(frow, _NPAD_SC)], fx)
    pltpu.sync_copy(ay_h.at[pl.ds(frow, _NPAD_SC)], fy)
    pltpu.sync_copy(az_h.at[pl.ds(frow, _NPAD_SC)], fz)
    s2, c2 = _sc_pass(cx, cy, cz, fx, fy, fz)
    ost[pl.ds(0, 16)] = s1
    ost[pl.ds(16, 16)] = c1
    ost[pl.ds(32, 16)] = s2
    ost[pl.ds(48, 16)] = c2
    pltpu.sync_copy(ost, out_h.at[pl.ds(w * 64, 64)])


def _sc_chamfer(ax, ay, az, bx, by, bz):
    mesh = plsc.VectorSubcoreMesh(core_axis_name="c", subcore_axis_name="s")
    run = functools.partial(
        pl.kernel,
        out_type=jax.ShapeDtypeStruct((32 * 64,), jnp.float32),
        mesh=mesh,
        scratch_types=[
            pltpu.VMEM((_CH,), jnp.float32),
            pltpu.VMEM((_CH,), jnp.float32),
            pltpu.VMEM((_CH,), jnp.float32),
            pltpu.VMEM((_NPAD_SC,), jnp.float32),
            pltpu.VMEM((_NPAD_SC,), jnp.float32),
            pltpu.VMEM((_NPAD_SC,), jnp.float32),
            pltpu.VMEM((64,), jnp.float32),
        ],
    )(_sc_body)
    partials = run(ax.reshape(-1), ay.reshape(-1), az.reshape(-1),
                   bx.reshape(-1), by.reshape(-1), bz.reshape(-1))
    p = partials.reshape(_NSC, _CPF, 4, 16).sum(axis=(1, 3))
    return p[:, 0] / p[:, 1] + p[:, 2] / p[:, 3]


def _sc_planes(plane):
    sub = plane[:_NSC]
    if _NPAD_SC <= NPAD:
        return sub[:, :_NPAD_SC]
    return jnp.pad(sub, ((0, 0), (0, _NPAD_SC - NPAD)),
                   constant_values=SENT)


def kernel(output, mask, target):
    B, T = output.shape[0], output.shape[1]
    out2 = output.reshape(BT, N)
    mask2 = mask.reshape(BT, N)
    tr = target[:, :, 0].reshape(BT, N)
    tx = target[:, :, 1].reshape(BT, N)
    ty = target[:, :, 2].reshape(BT, N)
    tz = target[:, :, 3].reshape(BT, N)
    ax, ay, az, bx, by, bz = _build_points(out2, mask2, tr, tx, ty, tz)
    parts = []
    if _NSC > 0:
        parts.append(_sc_chamfer(*(_sc_planes(p)
                                   for p in (ax, ay, az, bx, by, bz))))
    if _NSC < BT:
        parts.append(_pairwise(ax, ay, az, bx, by, bz).reshape(BT - _NSC))
    dc = jnp.concatenate(parts) if len(parts) > 1 else parts[0]
    ct = dc.reshape(T, B)
    return (jnp.mean(ct, axis=1), ct)


# speedup vs baseline: 14.9673x; 1.0611x over previous
import functools

import numpy as np
import jax
import jax.numpy as jnp
from jax import lax
from jax.experimental import pallas as pl
from jax.experimental.pallas import tpu as pltpu
from jax.experimental.pallas import tpu_sc as plsc

H, W = 32, 64
N = H * W
NPAD = 2176
RB = 1088
NRB = NPAD // RB
BT = 8
FOV_UP_DEG, FOV_DOWN_DEG = 3.0, -25.0
MASK_THRESHOLD = 0.5
SENT = 1000.0


def _dirs_np():
    fov_up = FOV_UP_DEG * np.pi / 180.0
    fov_down = FOV_DOWN_DEG * np.pi / 180.0
    fov = abs(fov_up) + abs(fov_down)
    proj_y = (np.arange(H, dtype=np.float32) + 0.5) / H
    proj_x = (np.arange(W, dtype=np.float32) + 0.5) / W
    pitch = (1.0 - proj_y) * fov - abs(fov_down)
    yaw = (2.0 * proj_x - 1.0) * np.pi
    pitch = pitch[:, None]
    yaw = yaw[None, :]
    dx = np.cos(pitch) * np.cos(yaw)
    dy = np.cos(pitch) * np.sin(yaw)
    dz = np.sin(pitch) * np.ones_like(yaw)
    dirs = np.stack([np.broadcast_to(dx, (H, W)),
                     np.broadcast_to(dy, (H, W)),
                     np.broadcast_to(dz, (H, W))], axis=-1).astype(np.float32)
    return dirs.reshape(N, 3)


_DIRS = _dirs_np()


def _build_body(out_ref, mask_ref, tr_ref, tx_ref, ty_ref, tz_ref,
                dx_ref, dy_ref, dz_ref,
                ax_ref, ay_ref, az_ref, bx_ref, by_ref, bz_ref):
    r = jnp.where(mask_ref[...] > MASK_THRESHOLD, out_ref[...], -1.0)
    valid = r > 0.0
    ax = jnp.where(valid, r * dx_ref[...], SENT)
    ay = jnp.where(valid, r * dy_ref[...], SENT)
    az = jnp.where(valid, r * dz_ref[...], SENT)
    tvalid = tr_ref[...] >= 0.0
    bx = jnp.where(tvalid, tx_ref[...], SENT)
    by = jnp.where(tvalid, ty_ref[...], SENT)
    bz = jnp.where(tvalid, tz_ref[...], SENT)
    for dst, src in ((ax_ref, ax), (ay_ref, ay), (az_ref, az)):
        dst[:, :N, 0] = src
        dst[:, N:, 0] = jnp.full((BT, NPAD - N), SENT, jnp.float32)
    for dst, src in ((bx_ref, bx), (by_ref, by), (bz_ref, bz)):
        dst[:, 0, :N] = src
        dst[:, 0, N:] = jnp.full((BT, NPAD - N), SENT, jnp.float32)


def _pair_body(axc, ayc, azc, bxr, byr, bzr, out_ref, colmin, acc):
    rb = pl.program_id(1)
    a_x = axc[0]
    a_y = ayc[0]
    a_z = azc[0]
    b_x = bxr[0]
    b_y = byr[0]
    b_z = bzr[0]
    dx = a_x - b_x
    dy = a_y - b_y
    dz = a_z - b_z
    d = dx * dx + dy * dy + dz * dz
    rmin = jnp.min(d, axis=1)
    s1 = jnp.sum(rmin)
    c1 = jnp.sum((rmin > 0.0).astype(jnp.float32))
    cm = jnp.min(d, axis=0, keepdims=True)

    @pl.when(rb == 0)
    def _():
        colmin[...] = cm
        acc[0] = s1
        acc[1] = c1

    @pl.when(rb > 0)
    def _():
        colmin[...] = jnp.minimum(colmin[...], cm)
        acc[0] = acc[0] + s1
        acc[1] = acc[1] + c1

    @pl.when(rb == NRB - 1)
    def _():
        cmf = colmin[...]
        s2 = jnp.sum(cmf)
        c2 = jnp.sum((cmf > 0.0).astype(jnp.float32))
        out_ref[...] = jnp.full((1, 1, 1), acc[0] / acc[1] + s2 / c2,
                                jnp.float32)


def _build_points(out2, mask2, tr, tx, ty, tz):
    dx = _DIRS[:, 0].reshape(1, N)
    dy = _DIRS[:, 1].reshape(1, N)
    dz = _DIRS[:, 2].reshape(1, N)
    acol = jax.ShapeDtypeStruct((BT, NPAD, 1), jnp.float32)
    brow = jax.ShapeDtypeStruct((BT, 1, NPAD), jnp.float32)
    return pl.pallas_call(
        _build_body,
        out_shape=(acol, acol, acol, brow, brow, brow),
    )(out2, mask2, tr, tx, ty, tz,
      jnp.asarray(dx), jnp.asarray(dy), jnp.asarray(dz))


def _pairwise(ax, ay, az, bx, by, bz):
    nf = BT - _NSC
    a_spec = pl.BlockSpec((1, RB, 1), lambda f, rb: (f + _NSC, rb, 0))
    b_spec = pl.BlockSpec((1, 1, NPAD), lambda f, rb: (f + _NSC, 0, 0))
    return pl.pallas_call(
        _pair_body,
        grid=(nf, NRB),
        in_specs=[a_spec, a_spec, a_spec, b_spec, b_spec, b_spec],
        out_specs=pl.BlockSpec((1, 1, 1), lambda f, rb: (f, 0, 0)),
        out_shape=jax.ShapeDtypeStruct((nf, 1, 1), jnp.float32),
        scratch_shapes=[
            pltpu.VMEM((1, NPAD), jnp.float32),
            pltpu.SMEM((2,), jnp.float32),
        ],
    )(ax, ay, az, bx, by, bz)


_NSC = 0
_CPF = 32 // max(_NSC, 1)
_CH = {8: 528, 4: 272, 2: 144, 1: 80}.get(_NSC, 16)
_NPAD_SC = _CPF * _CH
_UB = {8: 3, 4: 1, 2: 3, 1: 5}.get(_NSC, 1)
_NBLK = _CH // (16 * _UB)
_NBV = _NPAD_SC // 16


_GDN = lax.GatherDimensionNumbers(
    offset_dims=(), collapsed_slice_dims=(0,), start_index_map=(0,))


def _lane_splat(v, l):
    idx = jnp.full((16, 1), l, jnp.int32)
    return lax.gather(v, idx, _GDN, (1,),
                      mode=lax.GatherScatterMode.PROMISE_IN_BOUNDS)


def _sc_pass(cx, cy, cz, fx, fy, fz):
    big = jnp.full((16,), 1e30, jnp.float32)

    def outer(blk, carry):
        sacc, cacc = carry
        base = blk * (16 * _UB)
        avs = [(cx[pl.ds(base + 16 * u, 16)],
                cy[pl.ds(base + 16 * u, 16)],
                cz[pl.ds(base + 16 * u, 16)]) for u in range(_UB)]

        def inner(j16, ms):
            jb = j16 * 16
            bxv = fx[pl.ds(jb, 16)]
            byv = fy[pl.ds(jb, 16)]
            bzv = fz[pl.ds(jb, 16)]
            out = list(ms)
            for l in range(16):
                bxs = _lane_splat(bxv, l)
                bys = _lane_splat(byv, l)
                bzs = _lane_splat(bzv, l)
                for u in range(_UB):
                    ddx = avs[u][0] - bxs
                    ddy = avs[u][1] - bys
                    ddz = avs[u][2] - bzs
                    dd = ddx * ddx + ddy * ddy + ddz * ddz
                    out[u] = jnp.minimum(out[u], dd)
            return tuple(out)

        ms = lax.fori_loop(0, _NBV, inner, (big,) * _UB)
        for u in range(_UB):
            sacc = sacc + ms[u]
            cacc = cacc + jnp.sign(ms[u])
        return sacc, cacc

    zero = jnp.zeros((16,), jnp.float32)
    return lax.fori_loop(0, _NBLK, outer, (zero, zero))


def _sc_body(ax_h, ay_h, az_h, bx_h, by_h, bz_h, out_h,
             cx, cy, cz, fx, fy, fz, ost):
    c = lax.axis_index("c")
    s = lax.axis_index("s")
    w = s * 2 + c
    f = w // _CPF
    q = w % _CPF
    frow = f * _NPAD_SC
    coff = frow + q * _CH
    pltpu.sync_copy(ax_h.at[pl.ds(coff, _CH)], cx)
    pltpu.sync_copy(ay_h.at[pl.ds(coff, _CH)], cy)
    pltpu.sync_copy(az_h.at[pl.ds(coff, _CH)], cz)
    pltpu.sync_copy(bx_h.at[pl.ds(frow, _NPAD_SC)], fx)
    pltpu.sync_copy(by_h.at[pl.ds(frow, _NPAD_SC)], fy)
    pltpu.sync_copy(bz_h.at[pl.ds(frow, _NPAD_SC)], fz)
    s1, c1 = _sc_pass(cx, cy, cz, fx, fy, fz)
    pltpu.sync_copy(bx_h.at[pl.ds(coff, _CH)], cx)
    pltpu.sync_copy(by_h.at[pl.ds(coff, _CH)], cy)
    pltpu.sync_copy(bz_h.at[pl.ds(coff, _CH)], cz)
    pltpu.sync_copy(ax_h.at[pl.ds(frow, _NPAD_SC)], fx)
    pltpu.sync_copy(ay_h.at[pl.ds(frow, _NPAD_SC)], fy)
    pltpu.sync_copy(az_h.at[pl.ds(frow, _NPAD_SC)], fz)
    s2, c2 = _sc_pass(cx, cy, cz, fx, fy, fz)
    ost[pl.ds(0, 16)] = s1
    ost[pl.ds(16, 16)] = c1
    ost[pl.ds(32, 16)] = s2
    ost[pl.ds(48, 16)] = c2
    pltpu.sync_copy(ost, out_h.at[pl.ds(w * 64, 64)])


def _sc_chamfer(ax, ay, az, bx, by, bz):
    mesh = plsc.VectorSubcoreMesh(core_axis_name="c", subcore_axis_name="s")
    run = functools.partial(
        pl.kernel,
        out_type=jax.ShapeDtypeStruct((32 * 64,), jnp.float32),
        mesh=mesh,
        scratch_types=[
            pltpu.VMEM((_CH,), jnp.float32),
            pltpu.VMEM((_CH,), jnp.float32),
            pltpu.VMEM((_CH,), jnp.float32),
            pltpu.VMEM((_NPAD_SC,), jnp.float32),
            pltpu.VMEM((_NPAD_SC,), jnp.float32),
            pltpu.VMEM((_NPAD_SC,), jnp.float32),
            pltpu.VMEM((64,), jnp.float32),
        ],
    )(_sc_body)
    partials = run(ax.reshape(-1), ay.reshape(-1), az.reshape(-1),
                   bx.reshape(-1), by.reshape(-1), bz.reshape(-1))
    p = partials.reshape(_NSC, _CPF, 4, 16).sum(axis=(1, 3))
    return p[:, 0] / p[:, 1] + p[:, 2] / p[:, 3]


def _sc_planes(plane):
    sub = plane[:_NSC]
    if _NPAD_SC <= NPAD:
        return sub[:, :_NPAD_SC]
    return jnp.pad(sub, ((0, 0), (0, _NPAD_SC - NPAD)),
                   constant_values=SENT)


def kernel(output, mask, target):
    B, T = output.shape[0], output.shape[1]
    out2 = output.reshape(BT, N)
    mask2 = mask.reshape(BT, N)
    tr = target[:, :, 0].reshape(BT, N)
    tx = target[:, :, 1].reshape(BT, N)
    ty = target[:, :, 2].reshape(BT, N)
    tz = target[:, :, 3].reshape(BT, N)
    ax, ay, az, bx, by, bz = _build_points(out2, mask2, tr, tx, ty, tz)
    parts = []
    if _NSC > 0:
        parts.append(_sc_chamfer(*(_sc_planes(p)
                                   for p in (ax, ay, az, bx, by, bz))))
    if _NSC < BT:
        parts.append(_pairwise(ax, ay, az, bx, by, bz).reshape(BT - _NSC))
    dc = jnp.concatenate(parts) if len(parts) > 1 else parts[0]
    ct = dc.reshape(T, B)
    return (jnp.mean(ct, axis=1), ct)
